# baseline (speedup 1.0000x reference)
import functools

import jax
import jax.numpy as jnp
from jax import lax
from jax.experimental import pallas as pl
from jax.experimental.pallas import tpu as pltpu
from jax.experimental.pallas import tpu_sc as plsc

B = 16384
V = 1000
D = 16
OUT = 10
EPS = 1e-5

NC = 2
NS = 16
NW = NC * NS
BPW = B // NW
LANES = 16
CHUNKS = BPW // LANES
GCH = 128
NG = BPW // GCH
APT = B // NS
ROWS = 64
PR = 10


def _splat16(x):
    return jnp.full((LANES,), x, jnp.float32)


def _spl(c):
    return jnp.full((LANES,), c, jnp.int32)


def _body(ids_hbm, ages_hbm, table_hbm, params_hbm,
          out_hbm, p_hbm,
          ids_v, a_v, ab_v, t_v, params_v, p_loc, rows_v, outc_v,
          st_v, allst_v, shst, sem):
    cid = lax.axis_index("c")
    sid = lax.axis_index("s")
    wid = sid * NC + cid
    base = wid * BPW
    iota = lax.iota(jnp.int32, LANES)

    base_t = jnp.minimum(sid * ROWS, V - ROWS)
    stage = [
        pltpu.async_copy(ages_hbm.at[pl.ds(sid * APT, APT)], a_v, sem),
        pltpu.async_copy(ages_hbm.at[pl.ds(base, BPW)], ab_v, sem),
        pltpu.async_copy(table_hbm.at[pl.ds(base_t, ROWS)], t_v, sem),
        pltpu.async_copy(params_hbm, params_v, sem),
    ] + [pltpu.async_copy(ids_hbm.at[pl.ds(base + r * GCH, GCH)], ids_v.at[r],
                          sem)
         for r in range(NG)]
    for c in stage:
        c.wait()

    def stat_step(i, carry):
        s1, s2 = carry
        v = a_v[pl.ds(i * LANES, LANES)]
        return s1 + v, s2 + v * v

    z16 = jnp.zeros((LANES,), jnp.float32)
    s1, s2 = lax.fori_loop(0, APT // LANES, stat_step, (z16, z16))
    st_v[pl.ds(0, LANES)] = s1
    st_v[pl.ds(LANES, LANES)] = s2
    pltpu.sync_copy(st_v, shst.at[pl.ds(sid * 2 * LANES, 2 * LANES)])

    w16 = [plsc.load_gather(params_v, [_spl(j), _spl(D)]) for j in range(OUT)]

    def p_chunk(ch, carry):
        v_loc = ch * LANES + iota
        feats = [jnp.maximum(plsc.load_gather(t_v, [v_loc, _spl(d)]), 0.0)
                 for d in range(D)]

        def p_col(j, carry2):
            jspl = jnp.full((LANES,), j, jnp.int32)
            wrow = plsc.load_gather(params_v, [jspl, iota])
            acc = plsc.load_gather(params_v, [_spl(PR), jspl])
            for d in range(D):
                acc = acc + feats[d] * wrow[d]
            plsc.store_scatter(p_loc, [v_loc, jspl], acc)
            return carry2

        lax.fori_loop(0, OUT, p_col, 0)
        return carry

    lax.fori_loop(0, ROWS // LANES, p_chunk, 0)

    @pl.when(sid < NS - 1)
    def _():
        pltpu.sync_copy(p_loc, p_hbm.at[pl.ds(sid * ROWS, ROWS)])

    @pl.when(sid == NS - 1)
    def _():
        off = NS * ROWS - V
        pltpu.sync_copy(p_loc.at[pl.ds(off, ROWS - off)],
                        p_hbm.at[pl.ds((NS - 1) * ROWS, ROWS - off)])

    plsc.subcore_barrier()

    copies = [pltpu.async_copy(p_hbm.at[ids_v.at[r]],
                               rows_v.at[pl.ds(r * GCH, GCH)], sem)
              for r in range(NG)]

    pltpu.sync_copy(shst, allst_v)

    def red_step(i, carry):
        s1, s2 = carry
        return (s1 + allst_v[pl.ds(i * 2 * LANES, LANES)],
                s2 + allst_v[pl.ds(i * 2 * LANES + LANES, LANES)])

    r1, r2 = lax.fori_loop(0, NS, red_step, (z16, z16))
    meanv = _splat16(jnp.sum(r1)) * (1.0 / B)
    varv = _splat16(jnp.sum(r2)) * (1.0 / B) - meanv * meanv
    xv = varv + EPS
    yv = plsc.bitcast(0x5F3759DF - (plsc.bitcast(xv, jnp.int32) >> 1),
                      jnp.float32)
    for _ in range(4):
        yv = yv * (1.5 - 0.5 * xv * yv * yv)
    gspl = plsc.load_gather(params_v, [_spl(PR), _spl(OUT)])
    bespl = plsc.load_gather(params_v, [_spl(PR), _spl(OUT + 1)])
    k1 = yv * gspl

    for c in copies:
        c.wait()

    def chunk(ci, carry):
        i_vec = ci * LANES + iota
        a_chunk = ab_v[pl.ds(ci * LANES, LANES)]
        s_chunk = jnp.maximum((a_chunk - meanv) * k1 + bespl, 0.0)
        for j in range(OUT):
            jv = jnp.full((LANES,), j, jnp.int32)
            g = plsc.load_gather(rows_v, [i_vec, jv])
            plsc.store_scatter(outc_v, [i_vec * OUT + jv], g + s_chunk * w16[j])
        return carry

    lax.fori_loop(0, CHUNKS, chunk, 0)
    pltpu.sync_copy(outc_v, out_hbm.at[pl.ds(wid * BPW * OUT, BPW * OUT)])


_sc_kernel = functools.partial(
    pl.kernel,
    mesh=plsc.VectorSubcoreMesh(core_axis_name="c", subcore_axis_name="s"),
    out_type=(jax.ShapeDtypeStruct((B * OUT,), jnp.float32),
              jax.ShapeDtypeStruct((V, D), jnp.float32)),
    compiler_params=pltpu.CompilerParams(needs_layout_passes=False,
                                         use_tc_tiling_on_sc=False),
    scratch_types=[
        pltpu.VMEM((NG, GCH), jnp.int32),
        pltpu.VMEM((APT,), jnp.float32),
        pltpu.VMEM((BPW,), jnp.float32),
        pltpu.VMEM((ROWS, 128), jnp.float32),
        pltpu.VMEM((16, 128), jnp.float32),
        pltpu.VMEM((ROWS, D), jnp.float32),
        pltpu.VMEM((BPW, D), jnp.float32),
        pltpu.VMEM((BPW * OUT,), jnp.float32),
        pltpu.VMEM((2 * LANES,), jnp.float32),
        pltpu.VMEM((NS * 2 * LANES,), jnp.float32),
        pltpu.VMEM_SHARED((NS * 2 * LANES,), jnp.float32),
        pltpu.SemaphoreType.DMA,
    ],
)(_body)


def kernel(customer_ids, ages, emb_table, bn_gamma, bn_beta, W, b):
    table128 = jnp.pad(emb_table, ((0, 0), (0, 128 - D)))
    row10 = jnp.concatenate([b, bn_gamma, bn_beta,
                             jnp.zeros((116,), jnp.float32)])
    params = jnp.concatenate([
        jnp.pad(W, ((0, 0), (0, 128 - (D + 1)))),
        row10[None],
        jnp.zeros((5, 128), jnp.float32),
    ], axis=0)
    outflat, _ = _sc_kernel(customer_ids, ages, table128, params)
    return outflat.reshape(B, OUT)


# speedup vs baseline: 1.2622x; 1.2622x over previous
import functools

import jax
import jax.numpy as jnp
from jax import lax
from jax.experimental import pallas as pl
from jax.experimental.pallas import tpu as pltpu
from jax.experimental.pallas import tpu_sc as plsc

B = 16384
V = 1000
D = 16
OUT = 10
EPS = 1e-5

NC = 2
NS = 16
NW = NC * NS
BPW = B // NW
LANES = 16
CHUNKS = BPW // LANES
GCH = 128
NG = BPW // GCH
APT = B // NS
ROWS = 64
PR = 10


def _splat16(x):
    return jnp.full((LANES,), x, jnp.float32)


def _spl(c):
    return jnp.full((LANES,), c, jnp.int32)


def _body(ids_hbm, ages_hbm, table_hbm, params_hbm,
          out_hbm, p_hbm,
          ids_v, a_v, ab_v, t_v, params_v, p_loc, rows_v, outc_v,
          st_v, allst_v, shst, sem, sem_out):
    cid = lax.axis_index("c")
    sid = lax.axis_index("s")
    wid = sid * NC + cid
    base = wid * BPW
    iota = lax.iota(jnp.int32, LANES)

    base_t = jnp.minimum(sid * ROWS, V - ROWS)
    stage = [
        pltpu.async_copy(ages_hbm.at[pl.ds(sid * APT, APT)], a_v, sem),
        pltpu.async_copy(ages_hbm.at[pl.ds(base, BPW)], ab_v, sem),
        pltpu.async_copy(table_hbm.at[pl.ds(base_t, ROWS)], t_v, sem),
        pltpu.async_copy(params_hbm, params_v, sem),
    ] + [pltpu.async_copy(ids_hbm.at[pl.ds(base + r * GCH, GCH)], ids_v.at[r],
                          sem)
         for r in range(NG)]
    for c in stage:
        c.wait()

    def stat_step(i, carry):
        s1, s2 = carry
        v = a_v[pl.ds(i * LANES, LANES)]
        return s1 + v, s2 + v * v

    z16 = jnp.zeros((LANES,), jnp.float32)
    s1, s2 = lax.fori_loop(0, APT // LANES, stat_step, (z16, z16))
    st_v[pl.ds(0, LANES)] = s1
    st_v[pl.ds(LANES, LANES)] = s2
    pltpu.sync_copy(st_v, shst.at[pl.ds(sid * 2 * LANES, 2 * LANES)])

    w16 = [plsc.load_gather(params_v, [_spl(j), _spl(D)]) for j in range(OUT)]

    def p_chunk(ch, carry):
        v_loc = ch * LANES + iota
        feats = [jnp.maximum(plsc.load_gather(t_v, [v_loc, _spl(d)]), 0.0)
                 for d in range(D)]

        def p_col(j, carry2):
            jspl = jnp.full((LANES,), j, jnp.int32)
            wrow = plsc.load_gather(params_v, [jspl, iota])
            acc = plsc.load_gather(params_v, [_spl(PR), jspl])
            for d in range(D):
                acc = acc + feats[d] * wrow[d]
            plsc.store_scatter(p_loc, [v_loc, jspl], acc)
            return carry2

        lax.fori_loop(0, OUT, p_col, 0)
        return carry

    lax.fori_loop(0, ROWS // LANES, p_chunk, 0)

    @pl.when(sid < NS - 1)
    def _():
        pltpu.sync_copy(p_loc, p_hbm.at[pl.ds(sid * ROWS, ROWS)])

    @pl.when(sid == NS - 1)
    def _():
        off = NS * ROWS - V
        pltpu.sync_copy(p_loc.at[pl.ds(off, ROWS - off)],
                        p_hbm.at[pl.ds((NS - 1) * ROWS, ROWS - off)])

    plsc.subcore_barrier()

    copies = [pltpu.async_copy(p_hbm.at[ids_v.at[r]],
                               rows_v.at[pl.ds(r * GCH, GCH)], sem)
              for r in range(NG)]

    pltpu.sync_copy(shst, allst_v)

    def red_step(i, carry):
        s1, s2 = carry
        return (s1 + allst_v[pl.ds(i * 2 * LANES, LANES)],
                s2 + allst_v[pl.ds(i * 2 * LANES + LANES, LANES)])

    r1, r2 = lax.fori_loop(0, NS, red_step, (z16, z16))
    meanv = _splat16(jnp.sum(r1)) * (1.0 / B)
    varv = _splat16(jnp.sum(r2)) * (1.0 / B) - meanv * meanv
    xv = varv + EPS
    yv = plsc.bitcast(0x5F3759DF - (plsc.bitcast(xv, jnp.int32) >> 1),
                      jnp.float32)
    for _ in range(4):
        yv = yv * (1.5 - 0.5 * xv * yv * yv)
    gspl = plsc.load_gather(params_v, [_spl(PR), _spl(OUT)])
    bespl = plsc.load_gather(params_v, [_spl(PR), _spl(OUT + 1)])
    k1 = yv * gspl

    for c in copies:
        c.wait()

    def chunk(ci, carry):
        i_vec = ci * LANES + iota
        a_chunk = ab_v[pl.ds(ci * LANES, LANES)]
        s_chunk = jnp.maximum((a_chunk - meanv) * k1 + bespl, 0.0)
        for j in range(OUT):
            jv = jnp.full((LANES,), j, jnp.int32)
            g = plsc.load_gather(rows_v, [i_vec, jv])
            plsc.store_scatter(outc_v, [i_vec, jv], g + s_chunk * w16[j])
        pltpu.async_copy(outc_v.at[pl.ds(ci * LANES, LANES)],
                         out_hbm.at[pl.ds(base + ci * LANES, LANES)], sem_out)
        return carry

    lax.fori_loop(0, CHUNKS, chunk, 0)
    pltpu.make_async_copy(out_hbm.at[pl.ds(base, BPW)], outc_v,
                          sem_out).wait()


_sc_kernel = functools.partial(
    pl.kernel,
    mesh=plsc.VectorSubcoreMesh(core_axis_name="c", subcore_axis_name="s"),
    out_type=(jax.ShapeDtypeStruct((B, 128), jnp.float32),
              jax.ShapeDtypeStruct((V, D), jnp.float32)),
    compiler_params=pltpu.CompilerParams(needs_layout_passes=False,
                                         use_tc_tiling_on_sc=False),
    scratch_types=[
        pltpu.VMEM((NG, GCH), jnp.int32),
        pltpu.VMEM((APT,), jnp.float32),
        pltpu.VMEM((BPW,), jnp.float32),
        pltpu.VMEM((ROWS, 128), jnp.float32),
        pltpu.VMEM((16, 128), jnp.float32),
        pltpu.VMEM((ROWS, D), jnp.float32),
        pltpu.VMEM((BPW, D), jnp.float32),
        pltpu.VMEM((BPW, 128), jnp.float32),
        pltpu.VMEM((2 * LANES,), jnp.float32),
        pltpu.VMEM((NS * 2 * LANES,), jnp.float32),
        pltpu.VMEM_SHARED((NS * 2 * LANES,), jnp.float32),
        pltpu.SemaphoreType.DMA,
        pltpu.SemaphoreType.DMA,
    ],
)(_body)


def kernel(customer_ids, ages, emb_table, bn_gamma, bn_beta, W, b):
    table128 = jnp.pad(emb_table, ((0, 0), (0, 128 - D)))
    row10 = jnp.concatenate([b, bn_gamma, bn_beta,
                             jnp.zeros((116,), jnp.float32)])
    params = jnp.concatenate([
        jnp.pad(W, ((0, 0), (0, 128 - (D + 1)))),
        row10[None],
        jnp.zeros((5, 128), jnp.float32),
    ], axis=0)
    out128, _ = _sc_kernel(customer_ids, ages, table128, params)
    return out128[:, :OUT]


# speedup vs baseline: 1.2629x; 1.0006x over previous
import functools

import jax
import jax.numpy as jnp
from jax import lax
from jax.experimental import pallas as pl
from jax.experimental.pallas import tpu as pltpu
from jax.experimental.pallas import tpu_sc as plsc

B = 16384
V = 1000
D = 16
OUT = 10
EPS = 1e-5

NC = 2
NS = 16
NW = NC * NS
BPW = B // NW
LANES = 16
CHUNKS = BPW // LANES
GCH = 128
NG = BPW // GCH
APT = B // NS
ROWS = 64
PR = 10


def _splat16(x):
    return jnp.full((LANES,), x, jnp.float32)


def _spl(c):
    return jnp.full((LANES,), c, jnp.int32)


def _body(ids_hbm, ages_hbm, table_hbm, params_hbm,
          out_hbm, p_hbm,
          ids_v, a_v, ab_v, t_v, params_v, p_loc, rows_v, outc_v,
          st_v, allst_v, shst, sem, sem_out):
    cid = lax.axis_index("c")
    sid = lax.axis_index("s")
    wid = sid * NC + cid
    base = wid * BPW
    iota = lax.iota(jnp.int32, LANES)

    base_t = jnp.minimum(sid * ROWS, V - ROWS)
    stage = [
        pltpu.async_copy(ages_hbm.at[pl.ds(sid * APT, APT)], a_v, sem),
        pltpu.async_copy(ages_hbm.at[pl.ds(base, BPW)], ab_v, sem),
        pltpu.async_copy(table_hbm.at[pl.ds(base_t, ROWS)], t_v, sem),
        pltpu.async_copy(params_hbm, params_v, sem),
    ] + [pltpu.async_copy(ids_hbm.at[pl.ds(base + r * GCH, GCH)], ids_v.at[r],
                          sem)
         for r in range(NG)]
    for c in stage:
        c.wait()

    def stat_step(i, carry):
        s1, s2 = carry
        v = a_v[pl.ds(i * LANES, LANES)]
        return s1 + v, s2 + v * v

    z16 = jnp.zeros((LANES,), jnp.float32)
    s1, s2 = lax.fori_loop(0, APT // LANES, stat_step, (z16, z16))
    st_v[pl.ds(0, LANES)] = s1
    st_v[pl.ds(LANES, LANES)] = s2
    pltpu.sync_copy(st_v, shst.at[pl.ds(sid * 2 * LANES, 2 * LANES)])

    w16 = [plsc.load_gather(params_v, [_spl(j), _spl(D)]) for j in range(OUT)]

    def p_chunk(ch, carry):
        v_loc = ch * LANES + iota
        feats = [jnp.maximum(plsc.load_gather(t_v, [v_loc, _spl(d)]), 0.0)
                 for d in range(D)]

        def p_col(j, carry2):
            jspl = jnp.full((LANES,), j, jnp.int32)
            wrow = plsc.load_gather(params_v, [jspl, iota])
            acc = plsc.load_gather(params_v, [_spl(PR), jspl])
            for d in range(D):
                acc = acc + feats[d] * wrow[d]
            plsc.store_scatter(p_loc, [v_loc, jspl], acc)
            return carry2

        lax.fori_loop(0, OUT, p_col, 0)
        return carry

    lax.fori_loop(0, ROWS // LANES, p_chunk, 0)

    @pl.when(sid < NS - 1)
    def _():
        pltpu.sync_copy(p_loc, p_hbm.at[pl.ds(sid * ROWS, ROWS)])

    @pl.when(sid == NS - 1)
    def _():
        off = NS * ROWS - V
        pltpu.sync_copy(p_loc.at[pl.ds(off, ROWS - off)],
                        p_hbm.at[pl.ds((NS - 1) * ROWS, ROWS - off)])

    plsc.subcore_barrier()

    copies = [pltpu.async_copy(p_hbm.at[ids_v.at[r]],
                               rows_v.at[pl.ds(r * GCH, GCH)], sem)
              for r in range(NG)]

    pltpu.sync_copy(shst, allst_v)

    def red_step(i, carry):
        s1, s2 = carry
        return (s1 + allst_v[pl.ds(i * 2 * LANES, LANES)],
                s2 + allst_v[pl.ds(i * 2 * LANES + LANES, LANES)])

    r1, r2 = lax.fori_loop(0, NS, red_step, (z16, z16))
    meanv = _splat16(jnp.sum(r1)) * (1.0 / B)
    varv = _splat16(jnp.sum(r2)) * (1.0 / B) - meanv * meanv
    xv = varv + EPS
    yv = plsc.bitcast(0x5F3759DF - (plsc.bitcast(xv, jnp.int32) >> 1),
                      jnp.float32)
    for _ in range(4):
        yv = yv * (1.5 - 0.5 * xv * yv * yv)
    gspl = plsc.load_gather(params_v, [_spl(PR), _spl(OUT)])
    bespl = plsc.load_gather(params_v, [_spl(PR), _spl(OUT + 1)])
    k1 = yv * gspl

    for c in copies:
        c.wait()

    def chunk(ci, carry):
        i_vec = ci * LANES + iota
        a_chunk = ab_v[pl.ds(ci * LANES, LANES)]
        s_chunk = jnp.maximum((a_chunk - meanv) * k1 + bespl, 0.0)
        for j in range(OUT):
            jv = jnp.full((LANES,), j, jnp.int32)
            g = plsc.load_gather(rows_v, [i_vec, jv])
            plsc.store_scatter(outc_v, [i_vec, jv], g + s_chunk * w16[j])
        pltpu.async_copy(outc_v.at[pl.ds(ci * LANES, LANES)],
                         out_hbm.at[pl.ds(base + ci * LANES, LANES)], sem_out)
        return carry

    lax.fori_loop(0, CHUNKS, chunk, 0)
    pltpu.make_async_copy(out_hbm.at[pl.ds(base, BPW)], outc_v,
                          sem_out).wait()


_sc_kernel = functools.partial(
    pl.kernel,
    mesh=plsc.VectorSubcoreMesh(core_axis_name="c", subcore_axis_name="s"),
    out_type=(jax.ShapeDtypeStruct((B, 128), jnp.float32),
              jax.ShapeDtypeStruct((V, D), jnp.float32)),
    compiler_params=pltpu.CompilerParams(needs_layout_passes=False,
                                         use_tc_tiling_on_sc=False,
                                         skip_device_barrier=True,
                                         disable_bounds_checks=True),
    scratch_types=[
        pltpu.VMEM((NG, GCH), jnp.int32),
        pltpu.VMEM((APT,), jnp.float32),
        pltpu.VMEM((BPW,), jnp.float32),
        pltpu.VMEM((ROWS, 128), jnp.float32),
        pltpu.VMEM((16, 128), jnp.float32),
        pltpu.VMEM((ROWS, D), jnp.float32),
        pltpu.VMEM((BPW, D), jnp.float32),
        pltpu.VMEM((BPW, 128), jnp.float32),
        pltpu.VMEM((2 * LANES,), jnp.float32),
        pltpu.VMEM((NS * 2 * LANES,), jnp.float32),
        pltpu.VMEM_SHARED((NS * 2 * LANES,), jnp.float32),
        pltpu.SemaphoreType.DMA,
        pltpu.SemaphoreType.DMA,
    ],
)(_body)


def kernel(customer_ids, ages, emb_table, bn_gamma, bn_beta, W, b):
    table128 = jnp.pad(emb_table, ((0, 0), (0, 128 - D)))
    row10 = jnp.concatenate([b, bn_gamma, bn_beta,
                             jnp.zeros((116,), jnp.float32)])
    params = jnp.concatenate([
        jnp.pad(W, ((0, 0), (0, 128 - (D + 1)))),
        row10[None],
        jnp.zeros((5, 128), jnp.float32),
    ], axis=0)
    out128, _ = _sc_kernel(customer_ids, ages, table128, params)
    return out128[:, :OUT]
